# baseline (speedup 1.0000x reference)
import functools

import numpy as np
import jax
import jax.numpy as jnp
from jax import lax
from jax.experimental import pallas as pl
from jax.experimental.pallas import tpu as pltpu
from jax.experimental.pallas import tpu_sc as plsc

VOCAB = 100000
D = 128
B = 16
TX = 512
TY = 2048
T_OUT = TX + TY

NC = 2
NS = 16
NW = NC * NS
W_T = TY // NW
NSLOT = 4
LOOKAHEAD = 2
XROWS_W = (B * TX) // NW
LANES = 16


def _sine_pe(length, dim):
    pos = np.arange(length, dtype=np.float32)[:, None]
    div = np.exp(np.arange(0, dim, 2, dtype=np.float32) * -(np.log(10000.0) / dim))
    pe = np.zeros((length, dim), dtype=np.float32)
    pe[:, 0::2] = np.sin(pos * div)
    pe[:, 1::2] = np.cos(pos * div)
    return pe


_PE = _sine_pe(TY, D)

_mesh = plsc.VectorSubcoreMesh(core_axis_name="c", subcore_axis_name="s")


@functools.partial(
    pl.kernel,
    out_type=jax.ShapeDtypeStruct((B, T_OUT, D), jnp.float32),
    mesh=_mesh,
    scratch_types=[
        pltpu.VMEM((B, W_T), jnp.int32),
        pltpu.VMEM((NSLOT, W_T, D), jnp.float32),
        pltpu.VMEM((W_T, D), jnp.float32),
        pltpu.VMEM((XROWS_W, D), jnp.float32),
        pltpu.VMEM((LANES,), jnp.float32),
        [pltpu.SemaphoreType.DMA] * NSLOT,
        [pltpu.SemaphoreType.DMA] * NSLOT,
        pltpu.SemaphoreType.DMA,
        pltpu.SemaphoreType.DMA,
        pltpu.SemaphoreType.DMA,
    ],
)
def _emb_kernel(x_hbm, y_hbm, table_hbm, pe_hbm, alpha_hbm, out_hbm,
                idx_v, rows_v, pe_v, x_v, alpha_v, gsems, osems, xsem, isem,
                psem):
    s = lax.axis_index("s")
    c = lax.axis_index("c")
    w = s * NC + c
    tw = w * W_T

    icps = []
    for bb in range(B):
        cp = pltpu.make_async_copy(
            y_hbm.at[bb, pl.ds(tw, W_T)], idx_v.at[bb], isem)
        cp.start()
        icps.append(cp)
    pcp = pltpu.make_async_copy(pe_hbm.at[pl.ds(tw, W_T)], pe_v, psem)
    pcp.start()
    xb = s
    xhalf = c
    xin = pltpu.make_async_copy(
        x_hbm.at[xb, pl.ds(xhalf * XROWS_W, XROWS_W)], x_v, xsem)
    xin.start()
    pltpu.sync_copy(alpha_hbm, alpha_v)
    aval = alpha_v[...]

    for cp in icps:
        cp.wait()

    xin.wait()
    xout = pltpu.make_async_copy(
        x_v, out_hbm.at[xb, pl.ds(xhalf * XROWS_W, XROWS_W)], xsem)
    xout.start()

    pcp.wait()

    def scale_body(r, carry):
        for j in range(D // LANES):
            sl = pl.ds(j * LANES, LANES)
            pe_v[r, sl] = pe_v[r, sl] * aval
        return carry

    lax.fori_loop(0, W_T, scale_body, 0)

    def gather_start(bb, slot):
        return pltpu.async_copy(
            table_hbm.at[idx_v.at[bb]], rows_v.at[slot], gsems[slot])

    def out_start(bb, slot):
        return pltpu.async_copy(
            rows_v.at[slot], out_hbm.at[bb, pl.ds(TX + tw, W_T)], osems[slot])

    g_cps = [None] * NSLOT
    o_cps = [None] * NSLOT
    for step in range(B + LOOKAHEAD):
        c_g = step
        c_a = step - LOOKAHEAD
        if c_g < B:
            sg = c_g % NSLOT
            if o_cps[sg] is not None:
                o_cps[sg].wait()
                o_cps[sg] = None
            g_cps[sg] = gather_start(c_g, sg)
        if 0 <= c_a < B:
            sa = c_a % NSLOT
            g_cps[sa].wait()

            def add_body(r, carry):
                for j in range(D // LANES):
                    sl = pl.ds(j * LANES, LANES)
                    plsc.addupdate(rows_v.at[sa, r, sl], pe_v[r, sl])
                return carry

            lax.fori_loop(0, W_T, add_body, 0)
            o_cps[sa] = out_start(c_a, sa)

    xout.wait()
    for cp in o_cps:
        if cp is not None:
            cp.wait()


def kernel(x, y, table, alpha):
    if y.dtype != jnp.int32:
        y = y.astype(jnp.int32)
    alpha_vec = jnp.broadcast_to(
        jnp.asarray(alpha, dtype=jnp.float32).reshape(()), (LANES,))
    return _emb_kernel(x, y, table, jnp.asarray(_PE), alpha_vec)


# speedup vs baseline: 1.0408x; 1.0408x over previous
import functools

import numpy as np
import jax
import jax.numpy as jnp
from jax import lax
from jax.experimental import pallas as pl
from jax.experimental.pallas import tpu as pltpu
from jax.experimental.pallas import tpu_sc as plsc

VOCAB = 100000
D = 128
B = 16
TX = 512
TY = 2048
T_OUT = TX + TY

NC = 2
NS = 16
NW = NC * NS
W_T = TY // NW
NSLOT = 4
LOOKAHEAD = 2
XROWS_W = (B * TX) // NW
LANES = 16


def _sine_pe(length, dim):
    pos = np.arange(length, dtype=np.float32)[:, None]
    div = np.exp(np.arange(0, dim, 2, dtype=np.float32) * -(np.log(10000.0) / dim))
    pe = np.zeros((length, dim), dtype=np.float32)
    pe[:, 0::2] = np.sin(pos * div)
    pe[:, 1::2] = np.cos(pos * div)
    return pe


_PE = _sine_pe(TY, D)

_mesh = plsc.VectorSubcoreMesh(core_axis_name="c", subcore_axis_name="s")


@functools.partial(
    pl.kernel,
    out_type=jax.ShapeDtypeStruct((B, T_OUT, D), jnp.float32),
    mesh=_mesh,
    scratch_types=[
        pltpu.VMEM((B, W_T), jnp.int32),
        pltpu.VMEM((NSLOT, W_T, D), jnp.float32),
        pltpu.VMEM((W_T, D), jnp.float32),
        pltpu.VMEM((XROWS_W, D), jnp.float32),
        [pltpu.SemaphoreType.DMA] * NSLOT,
        [pltpu.SemaphoreType.DMA] * NSLOT,
        pltpu.SemaphoreType.DMA,
        pltpu.SemaphoreType.DMA,
        pltpu.SemaphoreType.DMA,
    ],
)
def _emb_kernel(x_hbm, y_hbm, table_hbm, ape_hbm, out_hbm,
                idx_v, rows_v, pe_v, x_v, gsems, osems, xsem, isem, psem):
    s = lax.axis_index("s")
    c = lax.axis_index("c")
    w = s * NC + c
    tw = w * W_T

    icps = []
    for bb in range(B):
        cp = pltpu.make_async_copy(
            y_hbm.at[bb, pl.ds(tw, W_T)], idx_v.at[bb], isem)
        cp.start()
        icps.append(cp)
    pcp = pltpu.make_async_copy(ape_hbm.at[pl.ds(tw, W_T)], pe_v, psem)
    pcp.start()
    xb = s
    xhalf = c
    xin = pltpu.make_async_copy(
        x_hbm.at[xb, pl.ds(xhalf * XROWS_W, XROWS_W)], x_v, xsem)
    xin.start()

    def gather_start(bb, slot):
        return pltpu.async_copy(
            table_hbm.at[idx_v.at[bb]], rows_v.at[slot], gsems[slot])

    def out_start(bb, slot):
        return pltpu.async_copy(
            rows_v.at[slot], out_hbm.at[bb, pl.ds(TX + tw, W_T)], osems[slot])

    pe_ready = [False]

    g_cps = [None] * NSLOT
    o_cps = [None] * NSLOT
    for step in range(B + LOOKAHEAD):
        c_g = step
        c_a = step - LOOKAHEAD
        if c_g < B:
            sg = c_g % NSLOT
            if o_cps[sg] is not None:
                o_cps[sg].wait()
                o_cps[sg] = None
            icps[c_g].wait()
            g_cps[sg] = gather_start(c_g, sg)
        if 0 <= c_a < B:
            if not pe_ready[0]:
                pcp.wait()
                pe_ready[0] = True
            sa = c_a % NSLOT
            g_cps[sa].wait()

            def add_body(r, carry):
                for j in range(D // LANES):
                    sl = pl.ds(j * LANES, LANES)
                    plsc.addupdate(rows_v.at[sa, r, sl], pe_v[r, sl])
                return carry

            lax.fori_loop(0, W_T, add_body, 0)
            o_cps[sa] = out_start(c_a, sa)

    xin.wait()
    xout = pltpu.make_async_copy(
        x_v, out_hbm.at[xb, pl.ds(xhalf * XROWS_W, XROWS_W)], xsem)
    xout.start()
    xout.wait()
    for cp in o_cps:
        if cp is not None:
            cp.wait()


def kernel(x, y, table, alpha):
    if y.dtype != jnp.int32:
        y = y.astype(jnp.int32)
    ape = alpha * jnp.asarray(_PE)
    return _emb_kernel(x, y, table, ape)


# speedup vs baseline: 1.0508x; 1.0096x over previous
import functools

import numpy as np
import jax
import jax.numpy as jnp
from jax import lax
from jax.experimental import pallas as pl
from jax.experimental.pallas import tpu as pltpu
from jax.experimental.pallas import tpu_sc as plsc

VOCAB = 100000
D = 128
B = 16
TX = 512
TY = 2048
T_OUT = TX + TY

NC = 2
NS = 16
NW = NC * NS
W_T = TY // NW
NSLOT = 6
LOOKAHEAD = 4
XROWS_W = (B * TX) // NW
LANES = 16


def _sine_pe(length, dim):
    pos = np.arange(length, dtype=np.float32)[:, None]
    div = np.exp(np.arange(0, dim, 2, dtype=np.float32) * -(np.log(10000.0) / dim))
    pe = np.zeros((length, dim), dtype=np.float32)
    pe[:, 0::2] = np.sin(pos * div)
    pe[:, 1::2] = np.cos(pos * div)
    return pe


_PE = _sine_pe(TY, D)

_mesh = plsc.VectorSubcoreMesh(core_axis_name="c", subcore_axis_name="s")


@functools.partial(
    pl.kernel,
    out_type=jax.ShapeDtypeStruct((B, T_OUT, D), jnp.float32),
    mesh=_mesh,
    scratch_types=[
        pltpu.VMEM((B, W_T), jnp.int32),
        pltpu.VMEM((NSLOT, W_T, D), jnp.float32),
        pltpu.VMEM((W_T, D), jnp.float32),
        pltpu.VMEM((XROWS_W, D), jnp.float32),
        [pltpu.SemaphoreType.DMA] * NSLOT,
        [pltpu.SemaphoreType.DMA] * NSLOT,
        pltpu.SemaphoreType.DMA,
        pltpu.SemaphoreType.DMA,
        pltpu.SemaphoreType.DMA,
    ],
)
def _emb_kernel(x_hbm, y_hbm, table_hbm, ape_hbm, out_hbm,
                idx_v, rows_v, pe_v, x_v, gsems, osems, xsem, isem, psem):
    s = lax.axis_index("s")
    c = lax.axis_index("c")
    w = s * NC + c
    tw = w * W_T

    icps = []
    for bb in range(B):
        cp = pltpu.make_async_copy(
            y_hbm.at[bb, pl.ds(tw, W_T)], idx_v.at[bb], isem)
        cp.start()
        icps.append(cp)
    pcp = pltpu.make_async_copy(ape_hbm.at[pl.ds(tw, W_T)], pe_v, psem)
    pcp.start()
    xb = s
    xhalf = c
    xin = pltpu.make_async_copy(
        x_hbm.at[xb, pl.ds(xhalf * XROWS_W, XROWS_W)], x_v, xsem)
    xin.start()

    def gather_start(bb, slot):
        return pltpu.async_copy(
            table_hbm.at[idx_v.at[bb]], rows_v.at[slot], gsems[slot])

    def out_start(bb, slot):
        return pltpu.async_copy(
            rows_v.at[slot], out_hbm.at[bb, pl.ds(TX + tw, W_T)], osems[slot])

    pe_ready = [False]

    g_cps = [None] * NSLOT
    o_cps = [None] * NSLOT
    for step in range(B + LOOKAHEAD):
        c_g = step
        c_a = step - LOOKAHEAD
        if c_g < B:
            sg = c_g % NSLOT
            if o_cps[sg] is not None:
                o_cps[sg].wait()
                o_cps[sg] = None
            icps[c_g].wait()
            g_cps[sg] = gather_start(c_g, sg)
        if 0 <= c_a < B:
            if not pe_ready[0]:
                pcp.wait()
                pe_ready[0] = True
            sa = c_a % NSLOT
            g_cps[sa].wait()

            def add_body(i, carry):
                r = i * 2
                for rr in range(2):
                    for j in range(D // LANES):
                        sl = pl.ds(j * LANES, LANES)
                        plsc.addupdate(rows_v.at[sa, r + rr, sl],
                                       pe_v[r + rr, sl])
                return carry

            lax.fori_loop(0, W_T // 2, add_body, 0)
            o_cps[sa] = out_start(c_a, sa)

    xin.wait()
    xout = pltpu.make_async_copy(
        x_v, out_hbm.at[xb, pl.ds(xhalf * XROWS_W, XROWS_W)], xsem)
    xout.start()
    xout.wait()
    for cp in o_cps:
        if cp is not None:
            cp.wait()


def kernel(x, y, table, alpha):
    if y.dtype != jnp.int32:
        y = y.astype(jnp.int32)
    ape = alpha * jnp.asarray(_PE)
    return _emb_kernel(x, y, table, ape)
